# baseline (speedup 1.0000x reference)
import functools

import jax
import jax.numpy as jnp
from jax import lax
from jax.experimental import pallas as pl
from jax.experimental.pallas import tpu as pltpu
from jax.experimental.pallas import tpu_sc as plsc

N = 10000
E = 320000
D_IN = 128
D_EMB = 256
B = 512
H = D_EMB // 2
NS = 16
NC = 2
K = 128
EC = 160
G = 16
NG = EC // G
GRP = 16
EPAD = NS * EC * K
NROW = 10240
RPT = NROW // NS
RB = 128
GIDC = NROW // NS // K
GB = 640
CPT = B // NS
EPT = B // NS

_f32 = jnp.float32
_i32 = jnp.int32

_MESH = dict(core_axis_name="c", subcore_axis_name="s")


def _hist_call(dst4, gid3, ones128, zrows):
    @functools.partial(
        pl.kernel,
        out_type=(
            jax.ShapeDtypeStruct((NROW, K), _f32),
            jax.ShapeDtypeStruct((NROW, K), _f32),
            jax.ShapeDtypeStruct((GB, K), _f32),
        ),
        mesh=plsc.VectorSubcoreMesh(**_MESH),
        scratch_types=[
            pltpu.VMEM((EC // 2, K), _i32),
            pltpu.VMEM((K, K), _f32),
            pltpu.VMEM_SHARED((NROW, K), _f32),
            pltpu.VMEM_SHARED((GB, K), _f32),
        ],
    )
    def k(dst_hbm, gid_hbm, ones_hbm, z_hbm, dega_hbm, degb_hbm, cnt_hbm,
          idx_v, ones_v, acc_deg, acc_cnt):
        c = lax.axis_index("c")
        s = lax.axis_index("s")

        pltpu.sync_copy(ones_hbm, ones_v)
        pltpu.sync_copy(dst_hbm.at[c, s], idx_v)

        @pl.loop(0, RPT // RB)
        def _(i):
            pltpu.sync_copy(z_hbm, acc_deg.at[pl.ds(s * RPT + i * RB, RB)])

        @pl.when(c == 1)
        def _():
            pltpu.sync_copy(z_hbm.at[pl.ds(0, GB // NS)],
                            acc_cnt.at[pl.ds(s * (GB // NS), GB // NS)])

        plsc.subcore_barrier()

        @pl.loop(0, EC // 2)
        def _(j):
            pltpu.sync_copy(ones_v, acc_deg.at[idx_v.at[j]], add=True)

        @pl.when(c == 1)
        def _():
            pltpu.sync_copy(gid_hbm.at[s], idx_v.at[pl.ds(0, GIDC)])

            @pl.loop(0, GIDC)
            def _(j):
                pltpu.sync_copy(ones_v, acc_cnt.at[idx_v.at[j]], add=True)

        plsc.subcore_barrier()

        def deg_writeback(o_hbm):
            @pl.loop(0, RPT // RB)
            def _(i):
                r = s * RPT + i * RB
                pltpu.sync_copy(acc_deg.at[pl.ds(r, RB)],
                                o_hbm.at[pl.ds(r, RB)])

        @pl.when(c == 0)
        def _():
            deg_writeback(dega_hbm)

        @pl.when(c == 1)
        def _():
            deg_writeback(degb_hbm)
            r = s * (GB // NS)
            pltpu.sync_copy(acc_cnt.at[pl.ds(r, GB // NS)],
                            cnt_hbm.at[pl.ds(r, GB // NS)])

    return k(dst4, gid3, ones128, zrows)


def _spmm_call(ga, gb, src3, dst3, zrows):
    @functools.partial(
        pl.kernel,
        out_type=(
            jax.ShapeDtypeStruct((NROW, H), _f32),
            jax.ShapeDtypeStruct((NROW, H), _f32),
        ),
        mesh=plsc.VectorSubcoreMesh(**_MESH),
        scratch_types=[
            pltpu.VMEM((2, GRP, K), _i32),
            pltpu.VMEM((2, GRP, K), _i32),
            pltpu.VMEM((K, H), _f32),
            pltpu.VMEM((K, H), _f32),
            pltpu.VMEM_SHARED((NROW, H), _f32),
            pltpu.SemaphoreType.DMA,
            pltpu.SemaphoreType.DMA,
            pltpu.SemaphoreType.DMA,
            pltpu.SemaphoreType.DMA,
            pltpu.SemaphoreType.DMA,
            pltpu.SemaphoreType.DMA,
            pltpu.SemaphoreType.DMA,
        ],
    )
    def k(ga_hbm, gb_hbm, src_hbm, dst_hbm, z_hbm, oa_hbm, ob_hbm,
          src_v, dst_v, buf0, buf1, acc,
          gsem0, gsem1, ssem0, ssem1, isem0, isem1, wsem):
        c = lax.axis_index("c")
        s = lax.axis_index("s")

        zcp = [
            pltpu.async_copy(z_hbm, acc.at[pl.ds(s * RPT + i * RB, RB)], wsem)
            for i in range(RPT // RB)
        ]
        for cp in zcp:
            cp.wait()

        plsc.subcore_barrier()

        bufs = (buf0, buf1)
        gsems = (gsem0, gsem1)
        ssems = (ssem0, ssem1)
        NT = EC // GRP

        def stage(t):
            return [
                pltpu.async_copy(src_hbm.at[s, pl.ds(t * GRP, GRP)],
                                 src_v.at[t % 2], isem0),
                pltpu.async_copy(dst_hbm.at[s, pl.ds(t * GRP, GRP)],
                                 dst_v.at[t % 2], isem1),
            ]

        def run(g_hbm):
            icp = stage(0)
            for t in range(NT):
                p = t % 2
                for cp in icp:
                    cp.wait()
                if t + 1 < NT:
                    icp = stage(t + 1)
                sv, dv = src_v.at[p], dst_v.at[p]
                gcp = [
                    pltpu.async_copy(g_hbm.at[sv.at[b]], bufs[b], gsems[b])
                    for b in range(2)
                ]
                scp = [None, None]
                for j in range(GRP):
                    b = j % 2
                    gcp[b].wait()
                    scp[b] = pltpu.async_copy(bufs[b], acc.at[dv.at[j]],
                                              ssems[b], add=True)
                    if j + 2 < GRP:
                        scp[b].wait()
                        gcp[b] = pltpu.async_copy(g_hbm.at[sv.at[j + 2]],
                                                  bufs[b], gsems[b])
                scp[0].wait()
                scp[1].wait()

        @pl.when(c == 0)
        def _():
            run(ga_hbm)

        @pl.when(c == 1)
        def _():
            run(gb_hbm)

        plsc.subcore_barrier()

        def writeback(o_hbm):
            wcp = [
                pltpu.async_copy(acc.at[pl.ds(s * RPT + i * RB, RB)],
                                 o_hbm.at[pl.ds(s * RPT + i * RB, RB)], wsem)
                for i in range(RPT // RB)
            ]
            for cp in wcp:
                cp.wait()

        @pl.when(c == 0)
        def _():
            writeback(oa_hbm)

        @pl.when(c == 1)
        def _():
            writeback(ob_hbm)

    return k(ga, gb, src3, dst3, zrows)


def _pool_call(ha, hb, gid3, center2, cnt16, zrows):
    @functools.partial(
        pl.kernel,
        out_type=(
            jax.ShapeDtypeStruct((B, H), _f32),
            jax.ShapeDtypeStruct((B, H), _f32),
            jax.ShapeDtypeStruct((B, H), _f32),
            jax.ShapeDtypeStruct((B, H), _f32),
        ),
        mesh=plsc.VectorSubcoreMesh(**_MESH),
        scratch_types=[
            pltpu.VMEM((GIDC, K), _i32),
            pltpu.VMEM((EPT,), _i32),
            pltpu.VMEM((K, H), _f32),
            pltpu.VMEM((EPT, H), _f32),
            pltpu.VMEM((CPT, H), _f32),
            pltpu.VMEM((CPT, K), _f32),
            pltpu.VMEM_SHARED((GB, H), _f32),
        ],
    )
    def k(ha_hbm, hb_hbm, gid_hbm, ctr_hbm, cnt_hbm, z_hbm,
          pa_hbm, pb_hbm, ea_hbm, eb_hbm,
          gid_v, ctr_v, rows_v, emb_v, pool_v, cnt_v, acc):
        c = lax.axis_index("c")
        s = lax.axis_index("s")

        pltpu.sync_copy(gid_hbm.at[s], gid_v)
        pltpu.sync_copy(ctr_hbm.at[s], ctr_v)
        pltpu.sync_copy(z_hbm.at[pl.ds(0, GB // NS)],
                        acc.at[pl.ds(s * (GB // NS), GB // NS)])

        plsc.subcore_barrier()

        def accumulate(h_hbm, e_hbm):
            @pl.loop(0, GIDC)
            def _(i):
                pltpu.sync_copy(h_hbm.at[pl.ds(s * RPT + i * K, K)], rows_v)
                pltpu.sync_copy(rows_v, acc.at[gid_v.at[i]], add=True)

            pltpu.sync_copy(h_hbm.at[ctr_v], emb_v)
            pltpu.sync_copy(emb_v, e_hbm.at[pl.ds(s * EPT, EPT)])

        @pl.when(c == 0)
        def _():
            accumulate(ha_hbm, ea_hbm)

        @pl.when(c == 1)
        def _():
            accumulate(hb_hbm, eb_hbm)

        plsc.subcore_barrier()

        def divide(p_hbm):
            pltpu.sync_copy(acc.at[pl.ds(s * CPT, CPT)], pool_v)
            pltpu.sync_copy(cnt_hbm.at[pl.ds(s * CPT, CPT)], cnt_v)

            @pl.loop(0, CPT * (H // 16))
            def _(i):
                r = i // (H // 16)
                f = i % (H // 16)
                den = jnp.maximum(cnt_v[r, pl.ds(0, 16)], 1.0)
                sl = pl.ds(f * 16, 16)
                pool_v[r, sl] = pool_v[r, sl] / den

            pltpu.sync_copy(pool_v, p_hbm.at[pl.ds(s * CPT, CPT)])

        @pl.when(c == 0)
        def _():
            divide(pa_hbm)

        @pl.when(c == 1)
        def _():
            divide(pb_hbm)

    return k(ha, hb, gid3, center2, cnt16, zrows)


_R = 1024


def _tc1_call(featp, w1, dega, degb):
    def body(f_ref, w_ref, da_ref, db_ref, ga_ref, gb_ref, dv_ref):
        i = pl.program_id(0)
        rows = i * _R + lax.broadcasted_iota(_i32, (_R, 1), 0)
        mask = (rows < N).astype(_f32)
        deg = da_ref[:, 0:1] + db_ref[:, 0:1]
        dinv = lax.rsqrt(deg + 1.0) * mask
        h = jnp.dot(f_ref[...], w_ref[...], preferred_element_type=_f32)
        g = dinv * h
        ga_ref[...] = g[:, :H]
        gb_ref[...] = g[:, H:]
        dv_ref[...] = jnp.broadcast_to(dinv, (_R, H))

    out = jax.ShapeDtypeStruct((NROW, H), _f32)
    return pl.pallas_call(
        body,
        grid=(NROW // _R,),
        in_specs=[
            pl.BlockSpec((_R, D_IN), lambda i: (i, 0)),
            pl.BlockSpec((D_IN, D_EMB), lambda i: (0, 0)),
            pl.BlockSpec((_R, K), lambda i: (i, 0)),
            pl.BlockSpec((_R, K), lambda i: (i, 0)),
        ],
        out_specs=[pl.BlockSpec((_R, H), lambda i: (i, 0))] * 3,
        out_shape=(out, out, out),
    )(featp, w1, dega, degb)


def _tc2_call(s1a, s1b, g1a, g1b, dinvb, w2, b1r):
    def body(sa_ref, sb_ref, ga_ref, gb_ref, dv_ref, w_ref, b_ref,
             oa_ref, ob_ref):
        d = dv_ref[...]
        h1a = jnp.maximum(d * (sa_ref[...] + ga_ref[...]) + b_ref[0:1, :H], 0.0)
        h1b = jnp.maximum(d * (sb_ref[...] + gb_ref[...]) + b_ref[0:1, H:], 0.0)
        h2 = (jnp.dot(h1a, w_ref[:H, :], preferred_element_type=_f32)
              + jnp.dot(h1b, w_ref[H:, :], preferred_element_type=_f32))
        oa_ref[...] = d * h2[:, :H]
        ob_ref[...] = d * h2[:, H:]

    out = jax.ShapeDtypeStruct((NROW, H), _f32)
    half = pl.BlockSpec((_R, H), lambda i: (i, 0))
    return pl.pallas_call(
        body,
        grid=(NROW // _R,),
        in_specs=[half, half, half, half, half,
                  pl.BlockSpec((D_EMB, D_EMB), lambda i: (0, 0)),
                  pl.BlockSpec((1, D_EMB), lambda i: (0, 0))],
        out_specs=[half, half],
        out_shape=(out, out),
    )(s1a, s1b, g1a, g1b, dinvb, w2, b1r)


def _tc3_call(s2a, s2b, g2a, g2b, dinvb, b2r):
    def body(sa_ref, sb_ref, ga_ref, gb_ref, dv_ref, b_ref, oa_ref, ob_ref):
        i = pl.program_id(0)
        rows = i * _R + lax.broadcasted_iota(_i32, (_R, 1), 0)
        mask = (rows < N).astype(_f32)
        d = dv_ref[...]
        oa_ref[...] = mask * (d * (sa_ref[...] + ga_ref[...]) + b_ref[0:1, :H])
        ob_ref[...] = mask * (d * (sb_ref[...] + gb_ref[...]) + b_ref[0:1, H:])

    out = jax.ShapeDtypeStruct((NROW, H), _f32)
    half = pl.BlockSpec((_R, H), lambda i: (i, 0))
    return pl.pallas_call(
        body,
        grid=(NROW // _R,),
        in_specs=[half, half, half, half, half,
                  pl.BlockSpec((1, D_EMB), lambda i: (0, 0))],
        out_specs=[half, half],
        out_shape=(out, out),
    )(s2a, s2b, g2a, g2b, dinvb, b2r)


def kernel(feat, edge_index, center_nids, graph_ids, W1, b1, W2, b2):
    src = edge_index[0].astype(_i32)
    dst = edge_index[1].astype(_i32)
    pad_e = N + (jnp.arange(EPAD - E, dtype=_i32) % (NROW - N))
    src3 = jnp.concatenate([src, pad_e]).reshape(NS, EC, K)
    dstp = jnp.concatenate([dst, pad_e])
    dst3 = dstp.reshape(NS, EC, K)
    dst4 = dstp.reshape(NC, NS, EC // 2, K)
    gid3 = jnp.concatenate(
        [graph_ids.astype(_i32), jnp.full((NROW - N,), B, _i32)]
    ).reshape(NS, GIDC, K)
    center2 = center_nids.astype(_i32).reshape(NS, EPT)
    featp = jnp.pad(feat, ((0, NROW - N), (0, 0)))
    zrows = jnp.zeros((RB, H), _f32)
    ones128 = jnp.ones((K, K), _f32)
    b1r = b1.reshape(1, D_EMB)
    b2r = b2.reshape(1, D_EMB)

    dega, degb, cnt128 = _hist_call(dst4, gid3, ones128, zrows)
    g1a, g1b, dinvb = _tc1_call(featp, W1, dega, degb)
    s1a, s1b = _spmm_call(g1a, g1b, src3, dst3, zrows)
    g2a, g2b = _tc2_call(s1a, s1b, g1a, g1b, dinvb, W2, b1r)
    s2a, s2b = _spmm_call(g2a, g2b, src3, dst3, zrows)
    ha, hb = _tc3_call(s2a, s2b, g2a, g2b, dinvb, b2r)
    pa, pb, ea, eb = _pool_call(ha, hb, gid3, center2, cnt128, zrows)

    emb = jnp.concatenate([ea, eb], axis=1)
    pooled = jnp.concatenate([pa, pb], axis=1)
    return (emb, pooled)


# speedup vs baseline: 1.0864x; 1.0864x over previous
import functools

import jax
import jax.numpy as jnp
from jax import lax
from jax.experimental import pallas as pl
from jax.experimental.pallas import tpu as pltpu
from jax.experimental.pallas import tpu_sc as plsc

N = 10000
E = 320000
D_IN = 128
D_EMB = 256
B = 512
H = D_EMB // 2
NS = 16
NC = 2
K = 128
EC = 160
G = 16
NG = EC // G
GRP = 16
EPAD = NS * EC * K
NROW = 10240
RPT = NROW // NS
RB = 128
GIDC = NROW // NS // K
GB = 640
CPT = B // NS
EPT = B // NS

_f32 = jnp.float32
_i32 = jnp.int32

_MESH = dict(core_axis_name="c", subcore_axis_name="s")


def _hist_call(dst4, gid3, ones128, zrows):
    @functools.partial(
        pl.kernel,
        out_type=(
            jax.ShapeDtypeStruct((NROW, K), _f32),
            jax.ShapeDtypeStruct((NROW, K), _f32),
            jax.ShapeDtypeStruct((GB, K), _f32),
        ),
        mesh=plsc.VectorSubcoreMesh(**_MESH),
        scratch_types=[
            pltpu.VMEM((EC // 2, K), _i32),
            pltpu.VMEM((K, K), _f32),
            pltpu.VMEM_SHARED((NROW, K), _f32),
            pltpu.VMEM_SHARED((GB, K), _f32),
        ],
    )
    def k(dst_hbm, gid_hbm, ones_hbm, z_hbm, dega_hbm, degb_hbm, cnt_hbm,
          idx_v, ones_v, acc_deg, acc_cnt):
        c = lax.axis_index("c")
        s = lax.axis_index("s")

        pltpu.sync_copy(ones_hbm, ones_v)
        pltpu.sync_copy(dst_hbm.at[c, s], idx_v)

        @pl.loop(0, RPT // RB)
        def _(i):
            pltpu.sync_copy(z_hbm, acc_deg.at[pl.ds(s * RPT + i * RB, RB)])

        @pl.when(c == 1)
        def _():
            pltpu.sync_copy(z_hbm.at[pl.ds(0, GB // NS)],
                            acc_cnt.at[pl.ds(s * (GB // NS), GB // NS)])

        plsc.subcore_barrier()

        @pl.loop(0, EC // 2)
        def _(j):
            pltpu.sync_copy(ones_v, acc_deg.at[idx_v.at[j]], add=True)

        @pl.when(c == 1)
        def _():
            pltpu.sync_copy(gid_hbm.at[s], idx_v.at[pl.ds(0, GIDC)])

            @pl.loop(0, GIDC)
            def _(j):
                pltpu.sync_copy(ones_v, acc_cnt.at[idx_v.at[j]], add=True)

        plsc.subcore_barrier()

        def deg_writeback(o_hbm):
            @pl.loop(0, RPT // RB)
            def _(i):
                r = s * RPT + i * RB
                pltpu.sync_copy(acc_deg.at[pl.ds(r, RB)],
                                o_hbm.at[pl.ds(r, RB)])

        @pl.when(c == 0)
        def _():
            deg_writeback(dega_hbm)

        @pl.when(c == 1)
        def _():
            deg_writeback(degb_hbm)
            r = s * (GB // NS)
            pltpu.sync_copy(acc_cnt.at[pl.ds(r, GB // NS)],
                            cnt_hbm.at[pl.ds(r, GB // NS)])

    return k(dst4, gid3, ones128, zrows)


def _spmm_call(ga, gb, src3, dst3, zrows):
    @functools.partial(
        pl.kernel,
        out_type=(
            jax.ShapeDtypeStruct((NROW, H), _f32),
            jax.ShapeDtypeStruct((NROW, H), _f32),
        ),
        mesh=plsc.VectorSubcoreMesh(**_MESH),
        scratch_types=[
            pltpu.VMEM((2, GRP, K), _i32),
            pltpu.VMEM((2, GRP, K), _i32),
            pltpu.VMEM((K // 2, H), _f32),
            pltpu.VMEM((K // 2, H), _f32),
            pltpu.VMEM((K // 2, H), _f32),
            pltpu.VMEM((K // 2, H), _f32),
            pltpu.VMEM_SHARED((NROW, H), _f32),
            pltpu.SemaphoreType.DMA,
            pltpu.SemaphoreType.DMA,
            pltpu.SemaphoreType.DMA,
            pltpu.SemaphoreType.DMA,
            pltpu.SemaphoreType.DMA,
            pltpu.SemaphoreType.DMA,
            pltpu.SemaphoreType.DMA,
            pltpu.SemaphoreType.DMA,
            pltpu.SemaphoreType.DMA,
            pltpu.SemaphoreType.DMA,
            pltpu.SemaphoreType.DMA,
        ],
    )
    def k(ga_hbm, gb_hbm, src_hbm, dst_hbm, z_hbm, oa_hbm, ob_hbm,
          src_v, dst_v, buf0, buf1, buf2, buf3, acc,
          gsem0, gsem1, gsem2, gsem3, ssem0, ssem1, ssem2, ssem3,
          isem0, isem1, wsem):
        c = lax.axis_index("c")
        s = lax.axis_index("s")

        zcp = [
            pltpu.async_copy(z_hbm, acc.at[pl.ds(s * RPT + i * RB, RB)], wsem)
            for i in range(RPT // RB)
        ]
        for cp in zcp:
            cp.wait()

        plsc.subcore_barrier()

        bufs = (buf0, buf1, buf2, buf3)
        gsems = (gsem0, gsem1, gsem2, gsem3)
        ssems = (ssem0, ssem1, ssem2, ssem3)
        NT = EC // GRP
        NB = 4
        U = 2 * GRP

        def stage(t):
            return [
                pltpu.async_copy(src_hbm.at[s, pl.ds(t * GRP, GRP)],
                                 src_v.at[t % 2], isem0),
                pltpu.async_copy(dst_hbm.at[s, pl.ds(t * GRP, GRP)],
                                 dst_v.at[t % 2], isem1),
            ]

        def run(g_hbm):
            icp = stage(0)
            for t in range(NT):
                p = t % 2
                for cp in icp:
                    cp.wait()
                if t + 1 < NT:
                    icp = stage(t + 1)
                sv, dv = src_v.at[p], dst_v.at[p]

                def half(iv, u):
                    return iv.at[u // 2, pl.ds((u % 2) * (K // 2), K // 2)]

                gcp = [
                    pltpu.async_copy(g_hbm.at[half(sv, b)], bufs[b], gsems[b])
                    for b in range(NB)
                ]
                scp = [None] * NB
                for u in range(U):
                    b = u % NB
                    gcp[b].wait()
                    scp[b] = pltpu.async_copy(bufs[b], acc.at[half(dv, u)],
                                              ssems[b], add=True)
                    if u + NB < U:
                        scp[b].wait()
                        gcp[b] = pltpu.async_copy(g_hbm.at[half(sv, u + NB)],
                                                  bufs[b], gsems[b])
                for b in range(NB):
                    scp[b].wait()

        @pl.when(c == 0)
        def _():
            run(ga_hbm)

        @pl.when(c == 1)
        def _():
            run(gb_hbm)

        plsc.subcore_barrier()

        def writeback(o_hbm):
            wcp = [
                pltpu.async_copy(acc.at[pl.ds(s * RPT + i * RB, RB)],
                                 o_hbm.at[pl.ds(s * RPT + i * RB, RB)], wsem)
                for i in range(RPT // RB)
            ]
            for cp in wcp:
                cp.wait()

        @pl.when(c == 0)
        def _():
            writeback(oa_hbm)

        @pl.when(c == 1)
        def _():
            writeback(ob_hbm)

    return k(ga, gb, src3, dst3, zrows)


def _pool_call(ha, hb, gid3, center2, cnt16, zrows):
    @functools.partial(
        pl.kernel,
        out_type=(
            jax.ShapeDtypeStruct((B, H), _f32),
            jax.ShapeDtypeStruct((B, H), _f32),
            jax.ShapeDtypeStruct((B, H), _f32),
            jax.ShapeDtypeStruct((B, H), _f32),
        ),
        mesh=plsc.VectorSubcoreMesh(**_MESH),
        scratch_types=[
            pltpu.VMEM((GIDC, K), _i32),
            pltpu.VMEM((EPT,), _i32),
            pltpu.VMEM((K, H), _f32),
            pltpu.VMEM((EPT, H), _f32),
            pltpu.VMEM((CPT, H), _f32),
            pltpu.VMEM((CPT, K), _f32),
            pltpu.VMEM_SHARED((GB, H), _f32),
        ],
    )
    def k(ha_hbm, hb_hbm, gid_hbm, ctr_hbm, cnt_hbm, z_hbm,
          pa_hbm, pb_hbm, ea_hbm, eb_hbm,
          gid_v, ctr_v, rows_v, emb_v, pool_v, cnt_v, acc):
        c = lax.axis_index("c")
        s = lax.axis_index("s")

        pltpu.sync_copy(gid_hbm.at[s], gid_v)
        pltpu.sync_copy(ctr_hbm.at[s], ctr_v)
        pltpu.sync_copy(z_hbm.at[pl.ds(0, GB // NS)],
                        acc.at[pl.ds(s * (GB // NS), GB // NS)])

        plsc.subcore_barrier()

        def accumulate(h_hbm, e_hbm):
            @pl.loop(0, GIDC)
            def _(i):
                pltpu.sync_copy(h_hbm.at[pl.ds(s * RPT + i * K, K)], rows_v)
                pltpu.sync_copy(rows_v, acc.at[gid_v.at[i]], add=True)

            pltpu.sync_copy(h_hbm.at[ctr_v], emb_v)
            pltpu.sync_copy(emb_v, e_hbm.at[pl.ds(s * EPT, EPT)])

        @pl.when(c == 0)
        def _():
            accumulate(ha_hbm, ea_hbm)

        @pl.when(c == 1)
        def _():
            accumulate(hb_hbm, eb_hbm)

        plsc.subcore_barrier()

        def divide(p_hbm):
            pltpu.sync_copy(acc.at[pl.ds(s * CPT, CPT)], pool_v)
            pltpu.sync_copy(cnt_hbm.at[pl.ds(s * CPT, CPT)], cnt_v)

            @pl.loop(0, CPT * (H // 16))
            def _(i):
                r = i // (H // 16)
                f = i % (H // 16)
                den = jnp.maximum(cnt_v[r, pl.ds(0, 16)], 1.0)
                sl = pl.ds(f * 16, 16)
                pool_v[r, sl] = pool_v[r, sl] / den

            pltpu.sync_copy(pool_v, p_hbm.at[pl.ds(s * CPT, CPT)])

        @pl.when(c == 0)
        def _():
            divide(pa_hbm)

        @pl.when(c == 1)
        def _():
            divide(pb_hbm)

    return k(ha, hb, gid3, center2, cnt16, zrows)


_R = 1024


def _tc1_call(featp, w1, dega, degb):
    def body(f_ref, w_ref, da_ref, db_ref, ga_ref, gb_ref, dv_ref):
        i = pl.program_id(0)
        rows = i * _R + lax.broadcasted_iota(_i32, (_R, 1), 0)
        mask = (rows < N).astype(_f32)
        deg = da_ref[:, 0:1] + db_ref[:, 0:1]
        dinv = lax.rsqrt(deg + 1.0) * mask
        h = jnp.dot(f_ref[...], w_ref[...], preferred_element_type=_f32)
        g = dinv * h
        ga_ref[...] = g[:, :H]
        gb_ref[...] = g[:, H:]
        dv_ref[...] = jnp.broadcast_to(dinv, (_R, H))

    out = jax.ShapeDtypeStruct((NROW, H), _f32)
    return pl.pallas_call(
        body,
        grid=(NROW // _R,),
        in_specs=[
            pl.BlockSpec((_R, D_IN), lambda i: (i, 0)),
            pl.BlockSpec((D_IN, D_EMB), lambda i: (0, 0)),
            pl.BlockSpec((_R, K), lambda i: (i, 0)),
            pl.BlockSpec((_R, K), lambda i: (i, 0)),
        ],
        out_specs=[pl.BlockSpec((_R, H), lambda i: (i, 0))] * 3,
        out_shape=(out, out, out),
    )(featp, w1, dega, degb)


def _tc2_call(s1a, s1b, g1a, g1b, dinvb, w2, b1r):
    def body(sa_ref, sb_ref, ga_ref, gb_ref, dv_ref, w_ref, b_ref,
             oa_ref, ob_ref):
        d = dv_ref[...]
        h1a = jnp.maximum(d * (sa_ref[...] + ga_ref[...]) + b_ref[0:1, :H], 0.0)
        h1b = jnp.maximum(d * (sb_ref[...] + gb_ref[...]) + b_ref[0:1, H:], 0.0)
        h2 = (jnp.dot(h1a, w_ref[:H, :], preferred_element_type=_f32)
              + jnp.dot(h1b, w_ref[H:, :], preferred_element_type=_f32))
        oa_ref[...] = d * h2[:, :H]
        ob_ref[...] = d * h2[:, H:]

    out = jax.ShapeDtypeStruct((NROW, H), _f32)
    half = pl.BlockSpec((_R, H), lambda i: (i, 0))
    return pl.pallas_call(
        body,
        grid=(NROW // _R,),
        in_specs=[half, half, half, half, half,
                  pl.BlockSpec((D_EMB, D_EMB), lambda i: (0, 0)),
                  pl.BlockSpec((1, D_EMB), lambda i: (0, 0))],
        out_specs=[half, half],
        out_shape=(out, out),
    )(s1a, s1b, g1a, g1b, dinvb, w2, b1r)


def _tc3_call(s2a, s2b, g2a, g2b, dinvb, b2r):
    def body(sa_ref, sb_ref, ga_ref, gb_ref, dv_ref, b_ref, oa_ref, ob_ref):
        i = pl.program_id(0)
        rows = i * _R + lax.broadcasted_iota(_i32, (_R, 1), 0)
        mask = (rows < N).astype(_f32)
        d = dv_ref[...]
        oa_ref[...] = mask * (d * (sa_ref[...] + ga_ref[...]) + b_ref[0:1, :H])
        ob_ref[...] = mask * (d * (sb_ref[...] + gb_ref[...]) + b_ref[0:1, H:])

    out = jax.ShapeDtypeStruct((NROW, H), _f32)
    half = pl.BlockSpec((_R, H), lambda i: (i, 0))
    return pl.pallas_call(
        body,
        grid=(NROW // _R,),
        in_specs=[half, half, half, half, half,
                  pl.BlockSpec((1, D_EMB), lambda i: (0, 0))],
        out_specs=[half, half],
        out_shape=(out, out),
    )(s2a, s2b, g2a, g2b, dinvb, b2r)


def kernel(feat, edge_index, center_nids, graph_ids, W1, b1, W2, b2):
    src = edge_index[0].astype(_i32)
    dst = edge_index[1].astype(_i32)
    pad_e = N + (jnp.arange(EPAD - E, dtype=_i32) % (NROW - N))
    src3 = jnp.concatenate([src, pad_e]).reshape(NS, EC, K)
    dstp = jnp.concatenate([dst, pad_e])
    dst3 = dstp.reshape(NS, EC, K)
    dst4 = dstp.reshape(NC, NS, EC // 2, K)
    gid3 = jnp.concatenate(
        [graph_ids.astype(_i32), jnp.full((NROW - N,), B, _i32)]
    ).reshape(NS, GIDC, K)
    center2 = center_nids.astype(_i32).reshape(NS, EPT)
    featp = jnp.pad(feat, ((0, NROW - N), (0, 0)))
    zrows = jnp.zeros((RB, H), _f32)
    ones128 = jnp.ones((K, K), _f32)
    b1r = b1.reshape(1, D_EMB)
    b2r = b2.reshape(1, D_EMB)

    dega, degb, cnt128 = _hist_call(dst4, gid3, ones128, zrows)
    g1a, g1b, dinvb = _tc1_call(featp, W1, dega, degb)
    s1a, s1b = _spmm_call(g1a, g1b, src3, dst3, zrows)
    g2a, g2b = _tc2_call(s1a, s1b, g1a, g1b, dinvb, W2, b1r)
    s2a, s2b = _spmm_call(g2a, g2b, src3, dst3, zrows)
    ha, hb = _tc3_call(s2a, s2b, g2a, g2b, dinvb, b2r)
    pa, pb, ea, eb = _pool_call(ha, hb, gid3, center2, cnt128, zrows)

    emb = jnp.concatenate([ea, eb], axis=1)
    pooled = jnp.concatenate([pa, pb], axis=1)
    return (emb, pooled)
